# Initial kernel scaffold; baseline (speedup 1.0000x reference)
import jax
import jax.numpy as jnp
from jax.experimental import pallas as pl


def kernel(x, edge_index, W1, b1, W2, b2):
    raise NotImplementedError("write your pallas kernel here")


# speedup vs baseline: 114.4845x; 114.4845x over previous
import functools

import jax
import jax.numpy as jnp
from jax import lax
from jax.experimental import pallas as pl
from jax.experimental.pallas import tpu as pltpu
from jax.experimental.pallas import tpu_sc as plsc

N = 100000
E = 3200000
ROWS = 784
NP = ROWS * 128
NC, NS = 2, 16
NW = NC * NS
EW = E // NW
C = 2000
NCH = EW // C
SL = NP // NS

_mesh = plsc.VectorSubcoreMesh(core_axis_name="c", subcore_axis_name="s")


def _ids():
    cid = lax.axis_index("c")
    sid = lax.axis_index("s")
    return cid, sid, sid * NC + cid


@functools.partial(
    pl.kernel,
    out_type=jax.ShapeDtypeStruct((NC, NP), jnp.float32),
    mesh=_mesh,
    scratch_types=[
        pltpu.VMEM((C,), jnp.int32),
        pltpu.VMEM((C,), jnp.float32),
        pltpu.VMEM_SHARED((NP,), jnp.float32),
    ],
)
def _deg_pass(dst_hbm, ones_hbm, zeros_hbm, out_hbm, idx_v, ones_v, acc_sh):
    cid, sid, wid = _ids()
    pltpu.sync_copy(zeros_hbm.at[pl.ds(sid * SL, SL)],
                    acc_sh.at[pl.ds(sid * SL, SL)])
    pltpu.sync_copy(ones_hbm, ones_v)
    plsc.subcore_barrier()

    def body(k, carry):
        base = wid * EW + k * C
        pltpu.sync_copy(dst_hbm.at[pl.ds(base, C)], idx_v)
        pltpu.sync_copy(ones_v, acc_sh.at[idx_v], add=True)
        return carry

    lax.fori_loop(0, NCH, body, 0)
    plsc.subcore_barrier()
    pltpu.sync_copy(acc_sh.at[pl.ds(sid * SL, SL)],
                    out_hbm.at[cid, pl.ds(sid * SL, SL)])


@functools.partial(
    pl.kernel,
    out_type=jax.ShapeDtypeStruct((NC, NP), jnp.float32),
    mesh=_mesh,
    scratch_types=[
        pltpu.VMEM((C,), jnp.int32),
        pltpu.VMEM((C,), jnp.int32),
        pltpu.VMEM((C,), jnp.float32),
        pltpu.VMEM_SHARED((NP,), jnp.float32),
        pltpu.SemaphoreType.DMA,
    ],
)
def _scalar_pass(src_hbm, dst_hbm, y_hbm, zeros_hbm, out_hbm,
                 idxs_v, idxd_v, vals_v, acc_sh, sem):
    cid, sid, wid = _ids()
    pltpu.sync_copy(zeros_hbm.at[pl.ds(sid * SL, SL)],
                    acc_sh.at[pl.ds(sid * SL, SL)])
    plsc.subcore_barrier()

    def body(k, carry):
        base = wid * EW + k * C
        pltpu.sync_copy(src_hbm.at[pl.ds(base, C)], idxs_v)
        pltpu.sync_copy(dst_hbm.at[pl.ds(base, C)], idxd_v)
        pltpu.async_copy(y_hbm.at[idxs_v], vals_v, sem).wait()
        pltpu.sync_copy(vals_v, acc_sh.at[idxd_v], add=True)
        return carry

    lax.fori_loop(0, NCH, body, 0)
    plsc.subcore_barrier()
    pltpu.sync_copy(acc_sh.at[pl.ds(sid * SL, SL)],
                    out_hbm.at[cid, pl.ds(sid * SL, SL)])


@functools.partial(
    pl.kernel,
    out_type=(jax.ShapeDtypeStruct((NC, NP), jnp.float32),
              jax.ShapeDtypeStruct((NC, NP), jnp.float32)),
    mesh=_mesh,
    scratch_types=[
        pltpu.VMEM((C,), jnp.int32),
        pltpu.VMEM((C,), jnp.int32),
        pltpu.VMEM((C,), jnp.float32),
        pltpu.VMEM((C,), jnp.float32),
        pltpu.VMEM_SHARED((NP,), jnp.float32),
        pltpu.VMEM_SHARED((NP,), jnp.float32),
        pltpu.SemaphoreType.DMA,
        pltpu.SemaphoreType.DMA,
    ],
)
def _pair_pass(src_hbm, dst_hbm, z0_hbm, z1_hbm, zeros_hbm,
               out0_hbm, out1_hbm,
               idxs_v, idxd_v, v0_v, v1_v, acc0_sh, acc1_sh, sem0, sem1):
    cid, sid, wid = _ids()
    pltpu.sync_copy(zeros_hbm.at[pl.ds(sid * SL, SL)],
                    acc0_sh.at[pl.ds(sid * SL, SL)])
    pltpu.sync_copy(zeros_hbm.at[pl.ds(sid * SL, SL)],
                    acc1_sh.at[pl.ds(sid * SL, SL)])
    plsc.subcore_barrier()

    def body(k, carry):
        base = wid * EW + k * C
        pltpu.sync_copy(src_hbm.at[pl.ds(base, C)], idxs_v)
        pltpu.sync_copy(dst_hbm.at[pl.ds(base, C)], idxd_v)
        cp0 = pltpu.async_copy(z0_hbm.at[idxs_v], v0_v, sem0)
        cp1 = pltpu.async_copy(z1_hbm.at[idxs_v], v1_v, sem1)
        cp0.wait()
        cp1.wait()
        pltpu.sync_copy(v0_v, acc0_sh.at[idxd_v], add=True)
        pltpu.sync_copy(v1_v, acc1_sh.at[idxd_v], add=True)
        return carry

    lax.fori_loop(0, NCH, body, 0)
    plsc.subcore_barrier()
    pltpu.sync_copy(acc0_sh.at[pl.ds(sid * SL, SL)],
                    out0_hbm.at[cid, pl.ds(sid * SL, SL)])
    pltpu.sync_copy(acc1_sh.at[pl.ds(sid * SL, SL)],
                    out1_hbm.at[cid, pl.ds(sid * SL, SL)])


def _tc1_body(degp_ref, x_ref, dinv_ref, y_ref):
    deg = degp_ref[0] + degp_ref[1] + 1.0
    dinv = lax.rsqrt(deg)
    dinv_ref[...] = dinv
    y_ref[...] = dinv * x_ref[...]


_tc1 = pl.pallas_call(
    _tc1_body,
    out_shape=(jax.ShapeDtypeStruct((ROWS, 128), jnp.float32),
               jax.ShapeDtypeStruct((ROWS, 128), jnp.float32)),
)


def _tc2_body(sp_ref, y_ref, dinv_ref, w1_ref, b1_ref, w2t_ref,
              z0_ref, z1_ref):
    dinv = dinv_ref[...]
    s = dinv * (sp_ref[0] + sp_ref[1] + y_ref[...])
    g0 = jnp.zeros_like(s)
    g1 = jnp.zeros_like(s)
    for j in range(16):
        h = jnp.maximum(s * w1_ref[0, j] + b1_ref[0, j], 0.0)
        g0 = g0 + h * w2t_ref[0, j]
        g1 = g1 + h * w2t_ref[1, j]
    z0_ref[...] = dinv * g0
    z1_ref[...] = dinv * g1


_tc2 = pl.pallas_call(
    _tc2_body,
    in_specs=[
        pl.BlockSpec(memory_space=pltpu.VMEM),
        pl.BlockSpec(memory_space=pltpu.VMEM),
        pl.BlockSpec(memory_space=pltpu.VMEM),
        pl.BlockSpec(memory_space=pltpu.SMEM),
        pl.BlockSpec(memory_space=pltpu.SMEM),
        pl.BlockSpec(memory_space=pltpu.SMEM),
    ],
    out_shape=(jax.ShapeDtypeStruct((ROWS, 128), jnp.float32),
               jax.ShapeDtypeStruct((ROWS, 128), jnp.float32)),
)


def _tc3_body(op0_ref, op1_ref, z0_ref, z1_ref, dinv_ref, b2_ref,
              out0_ref, out1_ref):
    dinv = dinv_ref[...]
    o0 = dinv * (op0_ref[0] + op0_ref[1] + z0_ref[...]) + b2_ref[0, 0]
    o1 = dinv * (op1_ref[0] + op1_ref[1] + z1_ref[...]) + b2_ref[0, 1]
    m = jnp.maximum(o0, o1)
    lse = m + jnp.log(jnp.exp(o0 - m) + jnp.exp(o1 - m))
    out0_ref[...] = o0 - lse
    out1_ref[...] = o1 - lse


_tc3 = pl.pallas_call(
    _tc3_body,
    in_specs=[
        pl.BlockSpec(memory_space=pltpu.VMEM),
        pl.BlockSpec(memory_space=pltpu.VMEM),
        pl.BlockSpec(memory_space=pltpu.VMEM),
        pl.BlockSpec(memory_space=pltpu.VMEM),
        pl.BlockSpec(memory_space=pltpu.VMEM),
        pl.BlockSpec(memory_space=pltpu.SMEM),
    ],
    out_shape=(jax.ShapeDtypeStruct((ROWS, 128), jnp.float32),
               jax.ShapeDtypeStruct((ROWS, 128), jnp.float32)),
)


def kernel(x, edge_index, W1, b1, W2, b2):
    ei = edge_index.astype(jnp.int32)
    src, dst = ei[0], ei[1]
    xp = jnp.zeros((NP,), jnp.float32).at[:N].set(x[:, 0])
    zeros = jnp.zeros((NP,), jnp.float32)
    ones = jnp.ones((C,), jnp.float32)

    degp = _deg_pass(dst, ones, zeros)
    dinv, y = _tc1(degp.reshape(NC, ROWS, 128), xp.reshape(ROWS, 128))

    sp = _scalar_pass(src, dst, y.reshape(NP), zeros)
    z0, z1 = _tc2(sp.reshape(NC, ROWS, 128), y, dinv,
                  W1, b1.reshape(1, 16), W2.T)

    op0, op1 = _pair_pass(src, dst, z0.reshape(NP), z1.reshape(NP), zeros)
    out0, out1 = _tc3(op0.reshape(NC, ROWS, 128), op1.reshape(NC, ROWS, 128),
                      z0, z1, dinv, b2.reshape(1, 2))

    return jnp.stack([out0.reshape(NP)[:N], out1.reshape(NP)[:N]], axis=1)


# speedup vs baseline: 133.3940x; 1.1652x over previous
import functools

import jax
import jax.numpy as jnp
from jax import lax
from jax.experimental import pallas as pl
from jax.experimental.pallas import tpu as pltpu
from jax.experimental.pallas import tpu_sc as plsc

N = 100000
E = 3200000
ROWS = 784
NP = ROWS * 128
NC, NS = 2, 16
NW = NC * NS
EW = E // NW
C = 20000
NCH = EW // C
SL = NP // NS

_mesh = plsc.VectorSubcoreMesh(core_axis_name="c", subcore_axis_name="s")


def _ids():
    cid = lax.axis_index("c")
    sid = lax.axis_index("s")
    return cid, sid, sid * NC + cid


@functools.partial(
    pl.kernel,
    out_type=jax.ShapeDtypeStruct((NC, NP), jnp.float32),
    mesh=_mesh,
    scratch_types=[
        pltpu.VMEM((C,), jnp.int32),
        pltpu.VMEM((C,), jnp.float32),
        pltpu.VMEM_SHARED((NP,), jnp.float32),
    ],
)
def _deg_pass(dst_hbm, ones_hbm, zeros_hbm, out_hbm, idx_v, ones_v, acc_sh):
    cid, sid, wid = _ids()
    pltpu.sync_copy(zeros_hbm.at[pl.ds(sid * SL, SL)],
                    acc_sh.at[pl.ds(sid * SL, SL)])
    pltpu.sync_copy(ones_hbm, ones_v)
    plsc.subcore_barrier()

    def body(k, carry):
        base = wid * EW + k * C
        pltpu.sync_copy(dst_hbm.at[pl.ds(base, C)], idx_v)
        pltpu.sync_copy(ones_v, acc_sh.at[idx_v], add=True)
        return carry

    lax.fori_loop(0, NCH, body, 0)
    plsc.subcore_barrier()
    pltpu.sync_copy(acc_sh.at[pl.ds(sid * SL, SL)],
                    out_hbm.at[cid, pl.ds(sid * SL, SL)])


@functools.partial(
    pl.kernel,
    out_type=jax.ShapeDtypeStruct((NC, NP), jnp.float32),
    mesh=_mesh,
    scratch_types=[
        pltpu.VMEM((C,), jnp.int32),
        pltpu.VMEM((C,), jnp.int32),
        pltpu.VMEM((C,), jnp.float32),
        pltpu.VMEM_SHARED((NP,), jnp.float32),
        pltpu.SemaphoreType.DMA,
    ],
)
def _scalar_pass(src_hbm, dst_hbm, y_hbm, zeros_hbm, out_hbm,
                 idxs_v, idxd_v, vals_v, acc_sh, sem):
    cid, sid, wid = _ids()
    pltpu.sync_copy(zeros_hbm.at[pl.ds(sid * SL, SL)],
                    acc_sh.at[pl.ds(sid * SL, SL)])
    plsc.subcore_barrier()

    def body(k, carry):
        base = wid * EW + k * C
        pltpu.sync_copy(src_hbm.at[pl.ds(base, C)], idxs_v)
        pltpu.sync_copy(dst_hbm.at[pl.ds(base, C)], idxd_v)
        pltpu.async_copy(y_hbm.at[idxs_v], vals_v, sem).wait()
        pltpu.sync_copy(vals_v, acc_sh.at[idxd_v], add=True)
        return carry

    lax.fori_loop(0, NCH, body, 0)
    plsc.subcore_barrier()
    pltpu.sync_copy(acc_sh.at[pl.ds(sid * SL, SL)],
                    out_hbm.at[cid, pl.ds(sid * SL, SL)])


@functools.partial(
    pl.kernel,
    out_type=(jax.ShapeDtypeStruct((NC, NP), jnp.float32),
              jax.ShapeDtypeStruct((NC, NP), jnp.float32)),
    mesh=_mesh,
    scratch_types=[
        pltpu.VMEM((C,), jnp.int32),
        pltpu.VMEM((C,), jnp.int32),
        pltpu.VMEM((C,), jnp.float32),
        pltpu.VMEM((C,), jnp.float32),
        pltpu.VMEM_SHARED((NP,), jnp.float32),
        pltpu.VMEM_SHARED((NP,), jnp.float32),
        pltpu.SemaphoreType.DMA,
        pltpu.SemaphoreType.DMA,
    ],
)
def _pair_pass(src_hbm, dst_hbm, z0_hbm, z1_hbm, zeros_hbm,
               out0_hbm, out1_hbm,
               idxs_v, idxd_v, v0_v, v1_v, acc0_sh, acc1_sh, sem0, sem1):
    cid, sid, wid = _ids()
    pltpu.sync_copy(zeros_hbm.at[pl.ds(sid * SL, SL)],
                    acc0_sh.at[pl.ds(sid * SL, SL)])
    pltpu.sync_copy(zeros_hbm.at[pl.ds(sid * SL, SL)],
                    acc1_sh.at[pl.ds(sid * SL, SL)])
    plsc.subcore_barrier()

    def body(k, carry):
        base = wid * EW + k * C
        pltpu.sync_copy(src_hbm.at[pl.ds(base, C)], idxs_v)
        pltpu.sync_copy(dst_hbm.at[pl.ds(base, C)], idxd_v)
        cp0 = pltpu.async_copy(z0_hbm.at[idxs_v], v0_v, sem0)
        cp1 = pltpu.async_copy(z1_hbm.at[idxs_v], v1_v, sem1)
        cp0.wait()
        cp1.wait()
        pltpu.sync_copy(v0_v, acc0_sh.at[idxd_v], add=True)
        pltpu.sync_copy(v1_v, acc1_sh.at[idxd_v], add=True)
        return carry

    lax.fori_loop(0, NCH, body, 0)
    plsc.subcore_barrier()
    pltpu.sync_copy(acc0_sh.at[pl.ds(sid * SL, SL)],
                    out0_hbm.at[cid, pl.ds(sid * SL, SL)])
    pltpu.sync_copy(acc1_sh.at[pl.ds(sid * SL, SL)],
                    out1_hbm.at[cid, pl.ds(sid * SL, SL)])


def _tc1_body(degp_ref, x_ref, dinv_ref, y_ref):
    deg = degp_ref[0] + degp_ref[1] + 1.0
    dinv = lax.rsqrt(deg)
    dinv_ref[...] = dinv
    y_ref[...] = dinv * x_ref[...]


_tc1 = pl.pallas_call(
    _tc1_body,
    out_shape=(jax.ShapeDtypeStruct((ROWS, 128), jnp.float32),
               jax.ShapeDtypeStruct((ROWS, 128), jnp.float32)),
)


def _tc2_body(sp_ref, y_ref, dinv_ref, w1_ref, b1_ref, w2t_ref,
              z0_ref, z1_ref):
    dinv = dinv_ref[...]
    s = dinv * (sp_ref[0] + sp_ref[1] + y_ref[...])
    g0 = jnp.zeros_like(s)
    g1 = jnp.zeros_like(s)
    for j in range(16):
        h = jnp.maximum(s * w1_ref[0, j] + b1_ref[0, j], 0.0)
        g0 = g0 + h * w2t_ref[0, j]
        g1 = g1 + h * w2t_ref[1, j]
    z0_ref[...] = dinv * g0
    z1_ref[...] = dinv * g1


_tc2 = pl.pallas_call(
    _tc2_body,
    in_specs=[
        pl.BlockSpec(memory_space=pltpu.VMEM),
        pl.BlockSpec(memory_space=pltpu.VMEM),
        pl.BlockSpec(memory_space=pltpu.VMEM),
        pl.BlockSpec(memory_space=pltpu.SMEM),
        pl.BlockSpec(memory_space=pltpu.SMEM),
        pl.BlockSpec(memory_space=pltpu.SMEM),
    ],
    out_shape=(jax.ShapeDtypeStruct((ROWS, 128), jnp.float32),
               jax.ShapeDtypeStruct((ROWS, 128), jnp.float32)),
)


def _tc3_body(op0_ref, op1_ref, z0_ref, z1_ref, dinv_ref, b2_ref,
              out0_ref, out1_ref):
    dinv = dinv_ref[...]
    o0 = dinv * (op0_ref[0] + op0_ref[1] + z0_ref[...]) + b2_ref[0, 0]
    o1 = dinv * (op1_ref[0] + op1_ref[1] + z1_ref[...]) + b2_ref[0, 1]
    m = jnp.maximum(o0, o1)
    lse = m + jnp.log(jnp.exp(o0 - m) + jnp.exp(o1 - m))
    out0_ref[...] = o0 - lse
    out1_ref[...] = o1 - lse


_tc3 = pl.pallas_call(
    _tc3_body,
    in_specs=[
        pl.BlockSpec(memory_space=pltpu.VMEM),
        pl.BlockSpec(memory_space=pltpu.VMEM),
        pl.BlockSpec(memory_space=pltpu.VMEM),
        pl.BlockSpec(memory_space=pltpu.VMEM),
        pl.BlockSpec(memory_space=pltpu.VMEM),
        pl.BlockSpec(memory_space=pltpu.SMEM),
    ],
    out_shape=(jax.ShapeDtypeStruct((ROWS, 128), jnp.float32),
               jax.ShapeDtypeStruct((ROWS, 128), jnp.float32)),
)


def kernel(x, edge_index, W1, b1, W2, b2):
    ei = edge_index.astype(jnp.int32)
    src, dst = ei[0], ei[1]
    xp = jnp.zeros((NP,), jnp.float32).at[:N].set(x[:, 0])
    zeros = jnp.zeros((NP,), jnp.float32)
    ones = jnp.ones((C,), jnp.float32)

    degp = _deg_pass(dst, ones, zeros)
    dinv, y = _tc1(degp.reshape(NC, ROWS, 128), xp.reshape(ROWS, 128))

    sp = _scalar_pass(src, dst, y.reshape(NP), zeros)
    z0, z1 = _tc2(sp.reshape(NC, ROWS, 128), y, dinv,
                  W1, b1.reshape(1, 16), W2.T)

    op0, op1 = _pair_pass(src, dst, z0.reshape(NP), z1.reshape(NP), zeros)
    out0, out1 = _tc3(op0.reshape(NC, ROWS, 128), op1.reshape(NC, ROWS, 128),
                      z0, z1, dinv, b2.reshape(1, 2))

    return jnp.stack([out0.reshape(NP)[:N], out1.reshape(NP)[:N]], axis=1)


# speedup vs baseline: 255.3818x; 1.9145x over previous
import functools

import jax
import jax.numpy as jnp
from jax import lax
from jax.experimental import pallas as pl
from jax.experimental.pallas import tpu as pltpu
from jax.experimental.pallas import tpu_sc as plsc

N = 100000
E = 3200000
ROWS = 784
NP = ROWS * 128
NC, NS = 2, 16
NW = NC * NS
EW = E // NW
C = 20000
NCH = EW // C
SL = NP // NS

_mesh = plsc.VectorSubcoreMesh(core_axis_name="c", subcore_axis_name="s")


def _ids():
    cid = lax.axis_index("c")
    sid = lax.axis_index("s")
    return cid, sid, sid * NC + cid


@functools.partial(
    pl.kernel,
    out_type=jax.ShapeDtypeStruct((NC, NP), jnp.float32),
    mesh=_mesh,
    scratch_types=[
        pltpu.VMEM((C,), jnp.int32),
        pltpu.VMEM((C,), jnp.float32),
        pltpu.VMEM_SHARED((NP,), jnp.float32),
    ],
)
def _deg_pass(dst_hbm, ones_hbm, zeros_hbm, out_hbm, idx_v, ones_v, acc_sh):
    cid, sid, wid = _ids()
    pltpu.sync_copy(zeros_hbm.at[pl.ds(sid * SL, SL)],
                    acc_sh.at[pl.ds(sid * SL, SL)])
    pltpu.sync_copy(ones_hbm, ones_v)
    plsc.subcore_barrier()

    def body(k, carry):
        base = wid * EW + k * C
        pltpu.sync_copy(dst_hbm.at[pl.ds(base, C)], idx_v)
        pltpu.sync_copy(ones_v, acc_sh.at[idx_v], add=True)
        return carry

    lax.fori_loop(0, NCH, body, 0)
    plsc.subcore_barrier()
    pltpu.sync_copy(acc_sh.at[pl.ds(sid * SL, SL)],
                    out_hbm.at[cid, pl.ds(sid * SL, SL)])


@functools.partial(
    pl.kernel,
    out_type=jax.ShapeDtypeStruct((NC, NP), jnp.float32),
    mesh=_mesh,
    scratch_types=[
        pltpu.VMEM((C,), jnp.int32),
        pltpu.VMEM((C,), jnp.int32),
        pltpu.VMEM((C,), jnp.float32),
        pltpu.VMEM_SHARED((NP,), jnp.float32),
        pltpu.VMEM_SHARED((NP,), jnp.float32),
        pltpu.SemaphoreType.DMA,
    ],
)
def _scalar_pass(src_hbm, dst_hbm, y_hbm, zeros_hbm, out_hbm,
                 idxs_v, idxd_v, vals_v, acc_sh, y_sh, sem):
    cid, sid, wid = _ids()
    pltpu.sync_copy(zeros_hbm.at[pl.ds(sid * SL, SL)],
                    acc_sh.at[pl.ds(sid * SL, SL)])
    pltpu.sync_copy(y_hbm.at[pl.ds(sid * SL, SL)],
                    y_sh.at[pl.ds(sid * SL, SL)])
    plsc.subcore_barrier()

    def body(k, carry):
        base = wid * EW + k * C
        pltpu.sync_copy(src_hbm.at[pl.ds(base, C)], idxs_v)
        pltpu.sync_copy(dst_hbm.at[pl.ds(base, C)], idxd_v)
        pltpu.async_copy(y_sh.at[idxs_v], vals_v, sem).wait()
        pltpu.sync_copy(vals_v, acc_sh.at[idxd_v], add=True)
        return carry

    lax.fori_loop(0, NCH, body, 0)
    plsc.subcore_barrier()
    pltpu.sync_copy(acc_sh.at[pl.ds(sid * SL, SL)],
                    out_hbm.at[cid, pl.ds(sid * SL, SL)])


@functools.partial(
    pl.kernel,
    out_type=(jax.ShapeDtypeStruct((NC, NP), jnp.float32),
              jax.ShapeDtypeStruct((NC, NP), jnp.float32)),
    mesh=_mesh,
    scratch_types=[
        pltpu.VMEM((C,), jnp.int32),
        pltpu.VMEM((C,), jnp.int32),
        pltpu.VMEM((C,), jnp.float32),
        pltpu.VMEM((C,), jnp.float32),
        pltpu.VMEM_SHARED((NP,), jnp.float32),
        pltpu.VMEM_SHARED((NP,), jnp.float32),
        pltpu.VMEM_SHARED((NP,), jnp.float32),
        pltpu.VMEM_SHARED((NP,), jnp.float32),
        pltpu.SemaphoreType.DMA,
        pltpu.SemaphoreType.DMA,
    ],
)
def _pair_pass(src_hbm, dst_hbm, z0_hbm, z1_hbm, zeros_hbm,
               out0_hbm, out1_hbm,
               idxs_v, idxd_v, v0_v, v1_v, acc0_sh, acc1_sh,
               z0_sh, z1_sh, sem0, sem1):
    cid, sid, wid = _ids()
    pltpu.sync_copy(zeros_hbm.at[pl.ds(sid * SL, SL)],
                    acc0_sh.at[pl.ds(sid * SL, SL)])
    pltpu.sync_copy(zeros_hbm.at[pl.ds(sid * SL, SL)],
                    acc1_sh.at[pl.ds(sid * SL, SL)])
    pltpu.sync_copy(z0_hbm.at[pl.ds(sid * SL, SL)],
                    z0_sh.at[pl.ds(sid * SL, SL)])
    pltpu.sync_copy(z1_hbm.at[pl.ds(sid * SL, SL)],
                    z1_sh.at[pl.ds(sid * SL, SL)])
    plsc.subcore_barrier()

    def body(k, carry):
        base = wid * EW + k * C
        pltpu.sync_copy(src_hbm.at[pl.ds(base, C)], idxs_v)
        pltpu.sync_copy(dst_hbm.at[pl.ds(base, C)], idxd_v)
        cp0 = pltpu.async_copy(z0_sh.at[idxs_v], v0_v, sem0)
        cp1 = pltpu.async_copy(z1_sh.at[idxs_v], v1_v, sem1)
        cp0.wait()
        cp1.wait()
        pltpu.sync_copy(v0_v, acc0_sh.at[idxd_v], add=True)
        pltpu.sync_copy(v1_v, acc1_sh.at[idxd_v], add=True)
        return carry

    lax.fori_loop(0, NCH, body, 0)
    plsc.subcore_barrier()
    pltpu.sync_copy(acc0_sh.at[pl.ds(sid * SL, SL)],
                    out0_hbm.at[cid, pl.ds(sid * SL, SL)])
    pltpu.sync_copy(acc1_sh.at[pl.ds(sid * SL, SL)],
                    out1_hbm.at[cid, pl.ds(sid * SL, SL)])


def _tc1_body(degp_ref, x_ref, dinv_ref, y_ref):
    deg = degp_ref[0] + degp_ref[1] + 1.0
    dinv = lax.rsqrt(deg)
    dinv_ref[...] = dinv
    y_ref[...] = dinv * x_ref[...]


_tc1 = pl.pallas_call(
    _tc1_body,
    out_shape=(jax.ShapeDtypeStruct((ROWS, 128), jnp.float32),
               jax.ShapeDtypeStruct((ROWS, 128), jnp.float32)),
)


def _tc2_body(sp_ref, y_ref, dinv_ref, w1_ref, b1_ref, w2t_ref,
              z0_ref, z1_ref):
    dinv = dinv_ref[...]
    s = dinv * (sp_ref[0] + sp_ref[1] + y_ref[...])
    g0 = jnp.zeros_like(s)
    g1 = jnp.zeros_like(s)
    for j in range(16):
        h = jnp.maximum(s * w1_ref[0, j] + b1_ref[0, j], 0.0)
        g0 = g0 + h * w2t_ref[0, j]
        g1 = g1 + h * w2t_ref[1, j]
    z0_ref[...] = dinv * g0
    z1_ref[...] = dinv * g1


_tc2 = pl.pallas_call(
    _tc2_body,
    in_specs=[
        pl.BlockSpec(memory_space=pltpu.VMEM),
        pl.BlockSpec(memory_space=pltpu.VMEM),
        pl.BlockSpec(memory_space=pltpu.VMEM),
        pl.BlockSpec(memory_space=pltpu.SMEM),
        pl.BlockSpec(memory_space=pltpu.SMEM),
        pl.BlockSpec(memory_space=pltpu.SMEM),
    ],
    out_shape=(jax.ShapeDtypeStruct((ROWS, 128), jnp.float32),
               jax.ShapeDtypeStruct((ROWS, 128), jnp.float32)),
)


def _tc3_body(op0_ref, op1_ref, z0_ref, z1_ref, dinv_ref, b2_ref,
              out0_ref, out1_ref):
    dinv = dinv_ref[...]
    o0 = dinv * (op0_ref[0] + op0_ref[1] + z0_ref[...]) + b2_ref[0, 0]
    o1 = dinv * (op1_ref[0] + op1_ref[1] + z1_ref[...]) + b2_ref[0, 1]
    m = jnp.maximum(o0, o1)
    lse = m + jnp.log(jnp.exp(o0 - m) + jnp.exp(o1 - m))
    out0_ref[...] = o0 - lse
    out1_ref[...] = o1 - lse


_tc3 = pl.pallas_call(
    _tc3_body,
    in_specs=[
        pl.BlockSpec(memory_space=pltpu.VMEM),
        pl.BlockSpec(memory_space=pltpu.VMEM),
        pl.BlockSpec(memory_space=pltpu.VMEM),
        pl.BlockSpec(memory_space=pltpu.VMEM),
        pl.BlockSpec(memory_space=pltpu.VMEM),
        pl.BlockSpec(memory_space=pltpu.SMEM),
    ],
    out_shape=(jax.ShapeDtypeStruct((ROWS, 128), jnp.float32),
               jax.ShapeDtypeStruct((ROWS, 128), jnp.float32)),
)


def kernel(x, edge_index, W1, b1, W2, b2):
    ei = edge_index.astype(jnp.int32)
    src, dst = ei[0], ei[1]
    xp = jnp.zeros((NP,), jnp.float32).at[:N].set(x[:, 0])
    zeros = jnp.zeros((NP,), jnp.float32)
    ones = jnp.ones((C,), jnp.float32)

    degp = _deg_pass(dst, ones, zeros)
    dinv, y = _tc1(degp.reshape(NC, ROWS, 128), xp.reshape(ROWS, 128))

    sp = _scalar_pass(src, dst, y.reshape(NP), zeros)
    z0, z1 = _tc2(sp.reshape(NC, ROWS, 128), y, dinv,
                  W1, b1.reshape(1, 16), W2.T)

    op0, op1 = _pair_pass(src, dst, z0.reshape(NP), z1.reshape(NP), zeros)
    out0, out1 = _tc3(op0.reshape(NC, ROWS, 128), op1.reshape(NC, ROWS, 128),
                      z0, z1, dinv, b2.reshape(1, 2))

    return jnp.stack([out0.reshape(NP)[:N], out1.reshape(NP)[:N]], axis=1)


# speedup vs baseline: 263.3903x; 1.0314x over previous
import functools

import jax
import jax.numpy as jnp
from jax import lax
from jax.experimental import pallas as pl
from jax.experimental.pallas import tpu as pltpu
from jax.experimental.pallas import tpu_sc as plsc

N = 100000
E = 3200000
ROWS = 784
NP = ROWS * 128
NC, NS = 2, 16
NW = NC * NS
EW = E // NW
C = 20000
NCH = EW // C
SL = NP // NS

_mesh = plsc.VectorSubcoreMesh(core_axis_name="c", subcore_axis_name="s")


def _ids():
    cid = lax.axis_index("c")
    sid = lax.axis_index("s")
    return cid, sid, sid * NC + cid


@functools.partial(
    pl.kernel,
    out_type=jax.ShapeDtypeStruct((NC, NP), jnp.float32),
    mesh=_mesh,
    scratch_types=[
        pltpu.VMEM((C,), jnp.int32),
        pltpu.VMEM((C,), jnp.float32),
        pltpu.VMEM_SHARED((NP,), jnp.float32),
    ],
)
def _deg_pass(dst_hbm, ones_hbm, zeros_hbm, out_hbm, idx_v, ones_v, acc_sh):
    cid, sid, wid = _ids()
    pltpu.sync_copy(zeros_hbm.at[pl.ds(sid * SL, SL)],
                    acc_sh.at[pl.ds(sid * SL, SL)])
    pltpu.sync_copy(ones_hbm, ones_v)
    plsc.subcore_barrier()

    def body(k, carry):
        base = wid * EW + k * C
        pltpu.sync_copy(dst_hbm.at[pl.ds(base, C)], idx_v)
        pltpu.sync_copy(ones_v, acc_sh.at[idx_v], add=True)
        return carry

    lax.fori_loop(0, NCH, body, 0)
    plsc.subcore_barrier()
    pltpu.sync_copy(acc_sh.at[pl.ds(sid * SL, SL)],
                    out_hbm.at[cid, pl.ds(sid * SL, SL)])


CP = 10000
NCHP = EW // CP


@functools.partial(
    pl.kernel,
    out_type=jax.ShapeDtypeStruct((NC, NP), jnp.float32),
    mesh=_mesh,
    scratch_types=[
        pltpu.VMEM((CP,), jnp.int32),
        pltpu.VMEM((CP,), jnp.int32),
        pltpu.VMEM((CP,), jnp.int32),
        pltpu.VMEM((CP,), jnp.int32),
        pltpu.VMEM((CP,), jnp.float32),
        pltpu.VMEM((CP,), jnp.float32),
        pltpu.VMEM_SHARED((NP,), jnp.float32),
        pltpu.VMEM_SHARED((NP,), jnp.float32),
        pltpu.SemaphoreType.DMA((2,)),
        pltpu.SemaphoreType.DMA((2,)),
        pltpu.SemaphoreType.DMA((2,)),
        pltpu.SemaphoreType.DMA((2,)),
    ],
)
def _scalar_pass(src_hbm, dst_hbm, y_hbm, zeros_hbm, out_hbm,
                 idxs0_v, idxs1_v, idxd0_v, idxd1_v, vals0_v, vals1_v,
                 acc_sh, y_sh,
                 sem_is, sem_id, sem_g, sem_sc):
    cid, sid, wid = _ids()
    idxs_v = [idxs0_v, idxs1_v]
    idxd_v = [idxd0_v, idxd1_v]
    vals_v = [vals0_v, vals1_v]

    def idx_start(c):
        b = c % 2
        base = wid * EW + c * CP
        cs = pltpu.async_copy(src_hbm.at[pl.ds(base, CP)], idxs_v[b],
                              sem_is.at[b])
        cd = pltpu.async_copy(dst_hbm.at[pl.ds(base, CP)], idxd_v[b],
                              sem_id.at[b])
        return cs, cd

    idx_cp = idx_start(0)
    pltpu.sync_copy(zeros_hbm.at[pl.ds(sid * SL, SL)],
                    acc_sh.at[pl.ds(sid * SL, SL)])
    pltpu.sync_copy(y_hbm.at[pl.ds(sid * SL, SL)],
                    y_sh.at[pl.ds(sid * SL, SL)])
    plsc.subcore_barrier()

    sc_cp = [None, None]
    for c in range(NCHP):
        b = c % 2
        idx_cp[0].wait()
        idx_cp[1].wait()
        if sc_cp[b] is not None:
            sc_cp[b].wait()
        g = pltpu.async_copy(y_sh.at[idxs_v[b]], vals_v[b], sem_g.at[b])
        g.wait()
        sc_cp[b] = pltpu.async_copy(vals_v[b], acc_sh.at[idxd_v[b]],
                                    sem_sc.at[b], add=True)
        if c + 1 < NCHP:
            if sc_cp[1 - b] is not None:
                sc_cp[1 - b].wait()
                sc_cp[1 - b] = None
            idx_cp = idx_start(c + 1)
    for cp in sc_cp:
        if cp is not None:
            cp.wait()
    plsc.subcore_barrier()
    pltpu.sync_copy(acc_sh.at[pl.ds(sid * SL, SL)],
                    out_hbm.at[cid, pl.ds(sid * SL, SL)])


@functools.partial(
    pl.kernel,
    out_type=(jax.ShapeDtypeStruct((NC, NP), jnp.float32),
              jax.ShapeDtypeStruct((NC, NP), jnp.float32)),
    mesh=_mesh,
    scratch_types=[
        pltpu.VMEM((C,), jnp.int32),
        pltpu.VMEM((C,), jnp.int32),
        pltpu.VMEM((C,), jnp.float32),
        pltpu.VMEM((C,), jnp.float32),
        pltpu.VMEM_SHARED((NP,), jnp.float32),
        pltpu.VMEM_SHARED((NP,), jnp.float32),
        pltpu.VMEM_SHARED((NP,), jnp.float32),
        pltpu.VMEM_SHARED((NP,), jnp.float32),
        pltpu.SemaphoreType.DMA,
        pltpu.SemaphoreType.DMA,
    ],
)
def _pair_pass(src_hbm, dst_hbm, z0_hbm, z1_hbm, zeros_hbm,
               out0_hbm, out1_hbm,
               idxs_v, idxd_v, v0_v, v1_v, acc0_sh, acc1_sh,
               z0_sh, z1_sh, sem0, sem1):
    cid, sid, wid = _ids()
    pltpu.sync_copy(zeros_hbm.at[pl.ds(sid * SL, SL)],
                    acc0_sh.at[pl.ds(sid * SL, SL)])
    pltpu.sync_copy(zeros_hbm.at[pl.ds(sid * SL, SL)],
                    acc1_sh.at[pl.ds(sid * SL, SL)])
    pltpu.sync_copy(z0_hbm.at[pl.ds(sid * SL, SL)],
                    z0_sh.at[pl.ds(sid * SL, SL)])
    pltpu.sync_copy(z1_hbm.at[pl.ds(sid * SL, SL)],
                    z1_sh.at[pl.ds(sid * SL, SL)])
    plsc.subcore_barrier()

    def body(k, carry):
        base = wid * EW + k * C
        pltpu.sync_copy(src_hbm.at[pl.ds(base, C)], idxs_v)
        pltpu.sync_copy(dst_hbm.at[pl.ds(base, C)], idxd_v)
        cp0 = pltpu.async_copy(z0_sh.at[idxs_v], v0_v, sem0)
        cp1 = pltpu.async_copy(z1_sh.at[idxs_v], v1_v, sem1)
        cp0.wait()
        cp1.wait()
        pltpu.sync_copy(v0_v, acc0_sh.at[idxd_v], add=True)
        pltpu.sync_copy(v1_v, acc1_sh.at[idxd_v], add=True)
        return carry

    lax.fori_loop(0, NCH, body, 0)
    plsc.subcore_barrier()
    pltpu.sync_copy(acc0_sh.at[pl.ds(sid * SL, SL)],
                    out0_hbm.at[cid, pl.ds(sid * SL, SL)])
    pltpu.sync_copy(acc1_sh.at[pl.ds(sid * SL, SL)],
                    out1_hbm.at[cid, pl.ds(sid * SL, SL)])


def _tc1_body(degp_ref, x_ref, dinv_ref, y_ref):
    deg = degp_ref[0] + degp_ref[1] + 1.0
    dinv = lax.rsqrt(deg)
    dinv_ref[...] = dinv
    y_ref[...] = dinv * x_ref[...]


_tc1 = pl.pallas_call(
    _tc1_body,
    out_shape=(jax.ShapeDtypeStruct((ROWS, 128), jnp.float32),
               jax.ShapeDtypeStruct((ROWS, 128), jnp.float32)),
)


def _tc2_body(sp_ref, y_ref, dinv_ref, w1_ref, b1_ref, w2t_ref,
              z0_ref, z1_ref):
    dinv = dinv_ref[...]
    s = dinv * (sp_ref[0] + sp_ref[1] + y_ref[...])
    g0 = jnp.zeros_like(s)
    g1 = jnp.zeros_like(s)
    for j in range(16):
        h = jnp.maximum(s * w1_ref[0, j] + b1_ref[0, j], 0.0)
        g0 = g0 + h * w2t_ref[0, j]
        g1 = g1 + h * w2t_ref[1, j]
    z0_ref[...] = dinv * g0
    z1_ref[...] = dinv * g1


_tc2 = pl.pallas_call(
    _tc2_body,
    in_specs=[
        pl.BlockSpec(memory_space=pltpu.VMEM),
        pl.BlockSpec(memory_space=pltpu.VMEM),
        pl.BlockSpec(memory_space=pltpu.VMEM),
        pl.BlockSpec(memory_space=pltpu.SMEM),
        pl.BlockSpec(memory_space=pltpu.SMEM),
        pl.BlockSpec(memory_space=pltpu.SMEM),
    ],
    out_shape=(jax.ShapeDtypeStruct((ROWS, 128), jnp.float32),
               jax.ShapeDtypeStruct((ROWS, 128), jnp.float32)),
)


def _tc3_body(op0_ref, op1_ref, z0_ref, z1_ref, dinv_ref, b2_ref,
              out0_ref, out1_ref):
    dinv = dinv_ref[...]
    o0 = dinv * (op0_ref[0] + op0_ref[1] + z0_ref[...]) + b2_ref[0, 0]
    o1 = dinv * (op1_ref[0] + op1_ref[1] + z1_ref[...]) + b2_ref[0, 1]
    m = jnp.maximum(o0, o1)
    lse = m + jnp.log(jnp.exp(o0 - m) + jnp.exp(o1 - m))
    out0_ref[...] = o0 - lse
    out1_ref[...] = o1 - lse


_tc3 = pl.pallas_call(
    _tc3_body,
    in_specs=[
        pl.BlockSpec(memory_space=pltpu.VMEM),
        pl.BlockSpec(memory_space=pltpu.VMEM),
        pl.BlockSpec(memory_space=pltpu.VMEM),
        pl.BlockSpec(memory_space=pltpu.VMEM),
        pl.BlockSpec(memory_space=pltpu.VMEM),
        pl.BlockSpec(memory_space=pltpu.SMEM),
    ],
    out_shape=(jax.ShapeDtypeStruct((ROWS, 128), jnp.float32),
               jax.ShapeDtypeStruct((ROWS, 128), jnp.float32)),
)


def kernel(x, edge_index, W1, b1, W2, b2):
    ei = edge_index.astype(jnp.int32)
    src, dst = ei[0], ei[1]
    xp = jnp.zeros((NP,), jnp.float32).at[:N].set(x[:, 0])
    zeros = jnp.zeros((NP,), jnp.float32)
    ones = jnp.ones((C,), jnp.float32)

    degp = _deg_pass(dst, ones, zeros)
    dinv, y = _tc1(degp.reshape(NC, ROWS, 128), xp.reshape(ROWS, 128))

    sp = _scalar_pass(src, dst, y.reshape(NP), zeros)
    z0, z1 = _tc2(sp.reshape(NC, ROWS, 128), y, dinv,
                  W1, b1.reshape(1, 16), W2.T)

    op0, op1 = _pair_pass(src, dst, z0.reshape(NP), z1.reshape(NP), zeros)
    out0, out1 = _tc3(op0.reshape(NC, ROWS, 128), op1.reshape(NC, ROWS, 128),
                      z0, z1, dinv, b2.reshape(1, 2))

    return jnp.stack([out0.reshape(NP)[:N], out1.reshape(NP)[:N]], axis=1)


# speedup vs baseline: 267.4007x; 1.0152x over previous
import functools

import jax
import jax.numpy as jnp
from jax import lax
from jax.experimental import pallas as pl
from jax.experimental.pallas import tpu as pltpu
from jax.experimental.pallas import tpu_sc as plsc

N = 100000
E = 3200000
ROWS = 784
NP = ROWS * 128
NC, NS = 2, 16
NW = NC * NS
EW = E // NW
C = 20000
NCH = EW // C
SL = NP // NS

_mesh = plsc.VectorSubcoreMesh(core_axis_name="c", subcore_axis_name="s")


def _ids():
    cid = lax.axis_index("c")
    sid = lax.axis_index("s")
    return cid, sid, sid * NC + cid


@functools.partial(
    pl.kernel,
    out_type=jax.ShapeDtypeStruct((NC, NP), jnp.float32),
    mesh=_mesh,
    scratch_types=[
        pltpu.VMEM((C,), jnp.int32),
        pltpu.VMEM((C,), jnp.float32),
        pltpu.VMEM_SHARED((NP,), jnp.float32),
    ],
)
def _deg_pass(ei_hbm, ones_hbm, zeros_hbm, out_hbm, idx_v, ones_v, acc_sh):
    cid, sid, wid = _ids()
    pltpu.sync_copy(zeros_hbm.at[pl.ds(sid * SL, SL)],
                    acc_sh.at[pl.ds(sid * SL, SL)])
    pltpu.sync_copy(ones_hbm, ones_v)
    plsc.subcore_barrier()

    def body(k, carry):
        base = wid * EW + k * C
        pltpu.sync_copy(ei_hbm.at[pl.ds(E + base, C)], idx_v)
        pltpu.sync_copy(ones_v, acc_sh.at[idx_v], add=True)
        return carry

    lax.fori_loop(0, NCH, body, 0)
    plsc.subcore_barrier()
    pltpu.sync_copy(acc_sh.at[pl.ds(sid * SL, SL)],
                    out_hbm.at[cid, pl.ds(sid * SL, SL)])


CP = 10000
NCHP = EW // CP


@functools.partial(
    pl.kernel,
    out_type=jax.ShapeDtypeStruct((NC, NP), jnp.float32),
    mesh=_mesh,
    scratch_types=[
        pltpu.VMEM((CP,), jnp.int32),
        pltpu.VMEM((CP,), jnp.int32),
        pltpu.VMEM((CP,), jnp.int32),
        pltpu.VMEM((CP,), jnp.int32),
        pltpu.VMEM((CP,), jnp.float32),
        pltpu.VMEM((CP,), jnp.float32),
        pltpu.VMEM_SHARED((NP,), jnp.float32),
        pltpu.VMEM_SHARED((NP,), jnp.float32),
        pltpu.SemaphoreType.DMA((2,)),
        pltpu.SemaphoreType.DMA((2,)),
        pltpu.SemaphoreType.DMA((2,)),
        pltpu.SemaphoreType.DMA((2,)),
    ],
)
def _scalar_pass(ei_hbm, y_hbm, zeros_hbm, out_hbm,
                 idxs0_v, idxs1_v, idxd0_v, idxd1_v, vals0_v, vals1_v,
                 acc_sh, y_sh,
                 sem_is, sem_id, sem_g, sem_sc):
    cid, sid, wid = _ids()
    idxs_v = [idxs0_v, idxs1_v]
    idxd_v = [idxd0_v, idxd1_v]
    vals_v = [vals0_v, vals1_v]

    def idx_start(c):
        b = c % 2
        base = wid * EW + c * CP
        cs = pltpu.async_copy(ei_hbm.at[pl.ds(base, CP)], idxs_v[b],
                              sem_is.at[b])
        cd = pltpu.async_copy(ei_hbm.at[pl.ds(E + base, CP)], idxd_v[b],
                              sem_id.at[b])
        return cs, cd

    idx_cp = idx_start(0)
    pltpu.sync_copy(zeros_hbm.at[pl.ds(sid * SL, SL)],
                    acc_sh.at[pl.ds(sid * SL, SL)])
    pltpu.sync_copy(y_hbm.at[pl.ds(sid * SL, SL)],
                    y_sh.at[pl.ds(sid * SL, SL)])
    plsc.subcore_barrier()

    sc_cp = [None, None]
    for c in range(NCHP):
        b = c % 2
        idx_cp[0].wait()
        idx_cp[1].wait()
        if sc_cp[b] is not None:
            sc_cp[b].wait()
        g = pltpu.async_copy(y_sh.at[idxs_v[b]], vals_v[b], sem_g.at[b])
        g.wait()
        sc_cp[b] = pltpu.async_copy(vals_v[b], acc_sh.at[idxd_v[b]],
                                    sem_sc.at[b], add=True)
        if c + 1 < NCHP:
            if sc_cp[1 - b] is not None:
                sc_cp[1 - b].wait()
                sc_cp[1 - b] = None
            idx_cp = idx_start(c + 1)
    for cp in sc_cp:
        if cp is not None:
            cp.wait()
    plsc.subcore_barrier()
    pltpu.sync_copy(acc_sh.at[pl.ds(sid * SL, SL)],
                    out_hbm.at[cid, pl.ds(sid * SL, SL)])


@functools.partial(
    pl.kernel,
    out_type=(jax.ShapeDtypeStruct((NC, NP), jnp.float32),
              jax.ShapeDtypeStruct((NC, NP), jnp.float32)),
    mesh=_mesh,
    scratch_types=[
        pltpu.VMEM((C,), jnp.int32),
        pltpu.VMEM((C,), jnp.int32),
        pltpu.VMEM((C,), jnp.float32),
        pltpu.VMEM((C,), jnp.float32),
        pltpu.VMEM_SHARED((NP,), jnp.float32),
        pltpu.VMEM_SHARED((NP,), jnp.float32),
        pltpu.VMEM_SHARED((NP,), jnp.float32),
        pltpu.VMEM_SHARED((NP,), jnp.float32),
        pltpu.SemaphoreType.DMA,
        pltpu.SemaphoreType.DMA,
    ],
)
def _pair_pass(ei_hbm, z0_hbm, z1_hbm, zeros_hbm,
               out0_hbm, out1_hbm,
               idxs_v, idxd_v, v0_v, v1_v, acc0_sh, acc1_sh,
               z0_sh, z1_sh, sem0, sem1):
    cid, sid, wid = _ids()
    pltpu.sync_copy(zeros_hbm.at[pl.ds(sid * SL, SL)],
                    acc0_sh.at[pl.ds(sid * SL, SL)])
    pltpu.sync_copy(zeros_hbm.at[pl.ds(sid * SL, SL)],
                    acc1_sh.at[pl.ds(sid * SL, SL)])
    pltpu.sync_copy(z0_hbm.at[pl.ds(sid * SL, SL)],
                    z0_sh.at[pl.ds(sid * SL, SL)])
    pltpu.sync_copy(z1_hbm.at[pl.ds(sid * SL, SL)],
                    z1_sh.at[pl.ds(sid * SL, SL)])
    plsc.subcore_barrier()

    def body(k, carry):
        base = wid * EW + k * C
        pltpu.sync_copy(ei_hbm.at[pl.ds(base, C)], idxs_v)
        pltpu.sync_copy(ei_hbm.at[pl.ds(E + base, C)], idxd_v)
        cp0 = pltpu.async_copy(z0_sh.at[idxs_v], v0_v, sem0)
        cp1 = pltpu.async_copy(z1_sh.at[idxs_v], v1_v, sem1)
        cp0.wait()
        cp1.wait()
        pltpu.sync_copy(v0_v, acc0_sh.at[idxd_v], add=True)
        pltpu.sync_copy(v1_v, acc1_sh.at[idxd_v], add=True)
        return carry

    lax.fori_loop(0, NCH, body, 0)
    plsc.subcore_barrier()
    pltpu.sync_copy(acc0_sh.at[pl.ds(sid * SL, SL)],
                    out0_hbm.at[cid, pl.ds(sid * SL, SL)])
    pltpu.sync_copy(acc1_sh.at[pl.ds(sid * SL, SL)],
                    out1_hbm.at[cid, pl.ds(sid * SL, SL)])


def _tc1_body(degp_ref, x_ref, dinv_ref, y_ref):
    deg = degp_ref[0] + degp_ref[1] + 1.0
    dinv = lax.rsqrt(deg)
    dinv_ref[...] = dinv
    y_ref[...] = dinv * x_ref[...]


_tc1 = pl.pallas_call(
    _tc1_body,
    out_shape=(jax.ShapeDtypeStruct((ROWS, 128), jnp.float32),
               jax.ShapeDtypeStruct((ROWS, 128), jnp.float32)),
)


def _tc2_body(sp_ref, y_ref, dinv_ref, w1_ref, b1_ref, w2t_ref,
              z0_ref, z1_ref):
    dinv = dinv_ref[...]
    s = dinv * (sp_ref[0] + sp_ref[1] + y_ref[...])
    g0 = jnp.zeros_like(s)
    g1 = jnp.zeros_like(s)
    for j in range(16):
        h = jnp.maximum(s * w1_ref[0, j] + b1_ref[0, j], 0.0)
        g0 = g0 + h * w2t_ref[0, j]
        g1 = g1 + h * w2t_ref[1, j]
    z0_ref[...] = dinv * g0
    z1_ref[...] = dinv * g1


_tc2 = pl.pallas_call(
    _tc2_body,
    in_specs=[
        pl.BlockSpec(memory_space=pltpu.VMEM),
        pl.BlockSpec(memory_space=pltpu.VMEM),
        pl.BlockSpec(memory_space=pltpu.VMEM),
        pl.BlockSpec(memory_space=pltpu.SMEM),
        pl.BlockSpec(memory_space=pltpu.SMEM),
        pl.BlockSpec(memory_space=pltpu.SMEM),
    ],
    out_shape=(jax.ShapeDtypeStruct((ROWS, 128), jnp.float32),
               jax.ShapeDtypeStruct((ROWS, 128), jnp.float32)),
)


def _tc3_body(op0_ref, op1_ref, z0_ref, z1_ref, dinv_ref, b2_ref,
              out0_ref, out1_ref):
    dinv = dinv_ref[...]
    o0 = dinv * (op0_ref[0] + op0_ref[1] + z0_ref[...]) + b2_ref[0, 0]
    o1 = dinv * (op1_ref[0] + op1_ref[1] + z1_ref[...]) + b2_ref[0, 1]
    m = jnp.maximum(o0, o1)
    lse = m + jnp.log(jnp.exp(o0 - m) + jnp.exp(o1 - m))
    out0_ref[...] = o0 - lse
    out1_ref[...] = o1 - lse


_tc3 = pl.pallas_call(
    _tc3_body,
    in_specs=[
        pl.BlockSpec(memory_space=pltpu.VMEM),
        pl.BlockSpec(memory_space=pltpu.VMEM),
        pl.BlockSpec(memory_space=pltpu.VMEM),
        pl.BlockSpec(memory_space=pltpu.VMEM),
        pl.BlockSpec(memory_space=pltpu.VMEM),
        pl.BlockSpec(memory_space=pltpu.SMEM),
    ],
    out_shape=(jax.ShapeDtypeStruct((ROWS, 128), jnp.float32),
               jax.ShapeDtypeStruct((ROWS, 128), jnp.float32)),
)


def kernel(x, edge_index, W1, b1, W2, b2):
    ei = edge_index.astype(jnp.int32).reshape(2 * E)
    xp = jnp.zeros((NP,), jnp.float32).at[:N].set(x[:, 0])
    zeros = jnp.zeros((NP,), jnp.float32)
    ones = jnp.ones((C,), jnp.float32)

    degp = _deg_pass(ei, ones, zeros)
    dinv, y = _tc1(degp.reshape(NC, ROWS, 128), xp.reshape(ROWS, 128))

    sp = _scalar_pass(ei, y.reshape(NP), zeros)
    z0, z1 = _tc2(sp.reshape(NC, ROWS, 128), y, dinv,
                  W1, b1.reshape(1, 16), W2.T)

    op0, op1 = _pair_pass(ei, z0.reshape(NP), z1.reshape(NP), zeros)
    out0, out1 = _tc3(op0.reshape(NC, ROWS, 128), op1.reshape(NC, ROWS, 128),
                      z0, z1, dinv, b2.reshape(1, 2))

    return jnp.stack([out0.reshape(NP)[:N], out1.reshape(NP)[:N]], axis=1)


# speedup vs baseline: 277.2646x; 1.0369x over previous
import functools

import jax
import jax.numpy as jnp
from jax import lax
from jax.experimental import pallas as pl
from jax.experimental.pallas import tpu as pltpu
from jax.experimental.pallas import tpu_sc as plsc

N = 100000
E = 3200000
ROWS = 784
NP = ROWS * 128
NC, NS = 2, 16
NW = NC * NS
EW = E // NW
SL = NP // NS

CD = 20000
NCHD = EW // CD
CP = 10000
NCHP = EW // CP

_mesh = plsc.VectorSubcoreMesh(core_axis_name="c", subcore_axis_name="s")


def _ids():
    cid = lax.axis_index("c")
    sid = lax.axis_index("s")
    return cid, sid, sid * NC + cid


@functools.partial(
    pl.kernel,
    out_type=jax.ShapeDtypeStruct((NC, NP), jnp.float32),
    mesh=_mesh,
    scratch_types=[
        pltpu.VMEM((CD,), jnp.int32),
        pltpu.VMEM((CD,), jnp.int32),
        pltpu.VMEM((CD,), jnp.float32),
        pltpu.VMEM_SHARED((NP,), jnp.float32),
        pltpu.SemaphoreType.DMA((2,)),
        pltpu.SemaphoreType.DMA((2,)),
    ],
)
def _deg_pass(ei_hbm, ones_hbm, zeros_hbm, out_hbm,
              idx0_v, idx1_v, ones_v, acc_sh, sem_i, sem_sc):
    cid, sid, wid = _ids()
    idx_v = [idx0_v, idx1_v]

    def idx_start(c):
        b = c % 2
        base = wid * EW + c * CD
        return pltpu.async_copy(ei_hbm.at[pl.ds(E + base, CD)], idx_v[b],
                                sem_i.at[b])

    icp = idx_start(0)
    pltpu.sync_copy(zeros_hbm.at[pl.ds(sid * SL, SL)],
                    acc_sh.at[pl.ds(sid * SL, SL)])
    pltpu.sync_copy(ones_hbm, ones_v)
    plsc.subcore_barrier()

    sc_cp = [None, None]
    for c in range(NCHD):
        b = c % 2
        icp.wait()
        if sc_cp[b] is not None:
            sc_cp[b].wait()
        sc_cp[b] = pltpu.async_copy(ones_v, acc_sh.at[idx_v[b]],
                                    sem_sc.at[b], add=True)
        if c + 1 < NCHD:
            if sc_cp[1 - b] is not None:
                sc_cp[1 - b].wait()
                sc_cp[1 - b] = None
            icp = idx_start(c + 1)
    for cp in sc_cp:
        if cp is not None:
            cp.wait()
    plsc.subcore_barrier()
    pltpu.sync_copy(acc_sh.at[pl.ds(sid * SL, SL)],
                    out_hbm.at[cid, pl.ds(sid * SL, SL)])


@functools.partial(
    pl.kernel,
    out_type=jax.ShapeDtypeStruct((NC, NP), jnp.float32),
    mesh=_mesh,
    scratch_types=[
        pltpu.VMEM((CP,), jnp.int32),
        pltpu.VMEM((CP,), jnp.int32),
        pltpu.VMEM((CP,), jnp.int32),
        pltpu.VMEM((CP,), jnp.int32),
        pltpu.VMEM((CP,), jnp.float32),
        pltpu.VMEM((CP,), jnp.float32),
        pltpu.VMEM_SHARED((NP,), jnp.float32),
        pltpu.VMEM_SHARED((NP,), jnp.float32),
        pltpu.SemaphoreType.DMA((2,)),
        pltpu.SemaphoreType.DMA((2,)),
        pltpu.SemaphoreType.DMA((2,)),
        pltpu.SemaphoreType.DMA((2,)),
    ],
)
def _scalar_pass(ei_hbm, y_hbm, zeros_hbm, out_hbm,
                 idxs0_v, idxs1_v, idxd0_v, idxd1_v, vals0_v, vals1_v,
                 acc_sh, y_sh,
                 sem_is, sem_id, sem_g, sem_sc):
    cid, sid, wid = _ids()
    idxs_v = [idxs0_v, idxs1_v]
    idxd_v = [idxd0_v, idxd1_v]
    vals_v = [vals0_v, vals1_v]

    def idx_start(c):
        b = c % 2
        base = wid * EW + c * CP
        cs = pltpu.async_copy(ei_hbm.at[pl.ds(base, CP)], idxs_v[b],
                              sem_is.at[b])
        cd = pltpu.async_copy(ei_hbm.at[pl.ds(E + base, CP)], idxd_v[b],
                              sem_id.at[b])
        return cs, cd

    idx_cp = idx_start(0)
    pltpu.sync_copy(zeros_hbm.at[pl.ds(sid * SL, SL)],
                    acc_sh.at[pl.ds(sid * SL, SL)])
    pltpu.sync_copy(y_hbm.at[pl.ds(sid * SL, SL)],
                    y_sh.at[pl.ds(sid * SL, SL)])
    plsc.subcore_barrier()

    sc_cp = [None, None]
    for c in range(NCHP):
        b = c % 2
        idx_cp[0].wait()
        idx_cp[1].wait()
        if sc_cp[b] is not None:
            sc_cp[b].wait()
        g = pltpu.async_copy(y_sh.at[idxs_v[b]], vals_v[b], sem_g.at[b])
        g.wait()
        sc_cp[b] = pltpu.async_copy(vals_v[b], acc_sh.at[idxd_v[b]],
                                    sem_sc.at[b], add=True)
        if c + 1 < NCHP:
            if sc_cp[1 - b] is not None:
                sc_cp[1 - b].wait()
                sc_cp[1 - b] = None
            idx_cp = idx_start(c + 1)
    for cp in sc_cp:
        if cp is not None:
            cp.wait()
    plsc.subcore_barrier()
    pltpu.sync_copy(acc_sh.at[pl.ds(sid * SL, SL)],
                    out_hbm.at[cid, pl.ds(sid * SL, SL)])


@functools.partial(
    pl.kernel,
    out_type=(jax.ShapeDtypeStruct((NC, NP), jnp.float32),
              jax.ShapeDtypeStruct((NC, NP), jnp.float32)),
    mesh=_mesh,
    scratch_types=[
        pltpu.VMEM((CP,), jnp.int32),
        pltpu.VMEM((CP,), jnp.int32),
        pltpu.VMEM((CP,), jnp.int32),
        pltpu.VMEM((CP,), jnp.int32),
        pltpu.VMEM((CP,), jnp.float32),
        pltpu.VMEM((CP,), jnp.float32),
        pltpu.VMEM((CP,), jnp.float32),
        pltpu.VMEM((CP,), jnp.float32),
        pltpu.VMEM_SHARED((NP,), jnp.float32),
        pltpu.VMEM_SHARED((NP,), jnp.float32),
        pltpu.VMEM_SHARED((NP,), jnp.float32),
        pltpu.VMEM_SHARED((NP,), jnp.float32),
        pltpu.SemaphoreType.DMA((2,)),
        pltpu.SemaphoreType.DMA((2,)),
        pltpu.SemaphoreType.DMA((2,)),
        pltpu.SemaphoreType.DMA((2,)),
        pltpu.SemaphoreType.DMA((2,)),
        pltpu.SemaphoreType.DMA((2,)),
    ],
)
def _pair_pass(ei_hbm, z01_hbm, zeros_hbm, out0_hbm, out1_hbm,
               idxs0_v, idxs1_v, idxd0_v, idxd1_v,
               v00_v, v01_v, v10_v, v11_v,
               acc0_sh, acc1_sh, z0_sh, z1_sh,
               sem_is, sem_id, sem_g0, sem_g1, sem_s0, sem_s1):
    cid, sid, wid = _ids()
    idxs_v = [idxs0_v, idxs1_v]
    idxd_v = [idxd0_v, idxd1_v]
    v0_v = [v00_v, v01_v]
    v1_v = [v10_v, v11_v]

    def idx_start(c):
        b = c % 2
        base = wid * EW + c * CP
        cs = pltpu.async_copy(ei_hbm.at[pl.ds(base, CP)], idxs_v[b],
                              sem_is.at[b])
        cd = pltpu.async_copy(ei_hbm.at[pl.ds(E + base, CP)], idxd_v[b],
                              sem_id.at[b])
        return cs, cd

    idx_cp = idx_start(0)
    pltpu.sync_copy(zeros_hbm.at[pl.ds(sid * SL, SL)],
                    acc0_sh.at[pl.ds(sid * SL, SL)])
    pltpu.sync_copy(zeros_hbm.at[pl.ds(sid * SL, SL)],
                    acc1_sh.at[pl.ds(sid * SL, SL)])
    pltpu.sync_copy(z01_hbm.at[pl.ds(sid * SL, SL)],
                    z0_sh.at[pl.ds(sid * SL, SL)])
    pltpu.sync_copy(z01_hbm.at[pl.ds(NP + sid * SL, SL)],
                    z1_sh.at[pl.ds(sid * SL, SL)])
    plsc.subcore_barrier()

    sc0 = [None, None]
    sc1 = [None, None]
    for c in range(NCHP):
        b = c % 2
        idx_cp[0].wait()
        idx_cp[1].wait()
        if sc0[b] is not None:
            sc0[b].wait()
            sc1[b].wait()
        g0 = pltpu.async_copy(z0_sh.at[idxs_v[b]], v0_v[b], sem_g0.at[b])
        g1 = pltpu.async_copy(z1_sh.at[idxs_v[b]], v1_v[b], sem_g1.at[b])
        g0.wait()
        g1.wait()
        sc0[b] = pltpu.async_copy(v0_v[b], acc0_sh.at[idxd_v[b]],
                                  sem_s0.at[b], add=True)
        sc1[b] = pltpu.async_copy(v1_v[b], acc1_sh.at[idxd_v[b]],
                                  sem_s1.at[b], add=True)
        if c + 1 < NCHP:
            if sc0[1 - b] is not None:
                sc0[1 - b].wait()
                sc1[1 - b].wait()
                sc0[1 - b] = None
                sc1[1 - b] = None
            idx_cp = idx_start(c + 1)
    for cp in sc0 + sc1:
        if cp is not None:
            cp.wait()
    plsc.subcore_barrier()
    pltpu.sync_copy(acc0_sh.at[pl.ds(sid * SL, SL)],
                    out0_hbm.at[cid, pl.ds(sid * SL, SL)])
    pltpu.sync_copy(acc1_sh.at[pl.ds(sid * SL, SL)],
                    out1_hbm.at[cid, pl.ds(sid * SL, SL)])


def _tc1_body(degp_ref, x_ref, dinv_ref, y_ref):
    deg = degp_ref[0] + degp_ref[1] + 1.0
    dinv = lax.rsqrt(deg)
    dinv_ref[...] = dinv
    y_ref[...] = dinv * x_ref[...]


_tc1 = pl.pallas_call(
    _tc1_body,
    out_shape=(jax.ShapeDtypeStruct((ROWS, 128), jnp.float32),
               jax.ShapeDtypeStruct((ROWS, 128), jnp.float32)),
)


def _tc2_body(sp_ref, y_ref, dinv_ref, w1_ref, b1_ref, w2t_ref, z_ref):
    dinv = dinv_ref[...]
    s = dinv * (sp_ref[0] + sp_ref[1] + y_ref[...])
    g0 = jnp.zeros_like(s)
    g1 = jnp.zeros_like(s)
    for j in range(16):
        h = jnp.maximum(s * w1_ref[0, j] + b1_ref[0, j], 0.0)
        g0 = g0 + h * w2t_ref[0, j]
        g1 = g1 + h * w2t_ref[1, j]
    z_ref[0] = dinv * g0
    z_ref[1] = dinv * g1


_tc2 = pl.pallas_call(
    _tc2_body,
    in_specs=[
        pl.BlockSpec(memory_space=pltpu.VMEM),
        pl.BlockSpec(memory_space=pltpu.VMEM),
        pl.BlockSpec(memory_space=pltpu.VMEM),
        pl.BlockSpec(memory_space=pltpu.SMEM),
        pl.BlockSpec(memory_space=pltpu.SMEM),
        pl.BlockSpec(memory_space=pltpu.SMEM),
    ],
    out_shape=jax.ShapeDtypeStruct((2, ROWS, 128), jnp.float32),
)


def _tc3_body(op0_ref, op1_ref, z_ref, dinv_ref, b2_ref, out0_ref, out1_ref):
    dinv = dinv_ref[...]
    o0 = dinv * (op0_ref[0] + op0_ref[1] + z_ref[0]) + b2_ref[0, 0]
    o1 = dinv * (op1_ref[0] + op1_ref[1] + z_ref[1]) + b2_ref[0, 1]
    m = jnp.maximum(o0, o1)
    lse = m + jnp.log(jnp.exp(o0 - m) + jnp.exp(o1 - m))
    out0_ref[...] = o0 - lse
    out1_ref[...] = o1 - lse


_tc3 = pl.pallas_call(
    _tc3_body,
    in_specs=[
        pl.BlockSpec(memory_space=pltpu.VMEM),
        pl.BlockSpec(memory_space=pltpu.VMEM),
        pl.BlockSpec(memory_space=pltpu.VMEM),
        pl.BlockSpec(memory_space=pltpu.VMEM),
        pl.BlockSpec(memory_space=pltpu.SMEM),
    ],
    out_shape=(jax.ShapeDtypeStruct((ROWS, 128), jnp.float32),
               jax.ShapeDtypeStruct((ROWS, 128), jnp.float32)),
)


def kernel(x, edge_index, W1, b1, W2, b2):
    ei = edge_index.astype(jnp.int32).reshape(2 * E)
    xp = jnp.zeros((NP,), jnp.float32).at[:N].set(x[:, 0])
    zeros = jnp.zeros((NP,), jnp.float32)
    ones = jnp.ones((CD,), jnp.float32)

    degp = _deg_pass(ei, ones, zeros)
    dinv, y = _tc1(degp.reshape(NC, ROWS, 128), xp.reshape(ROWS, 128))

    sp = _scalar_pass(ei, y.reshape(NP), zeros)
    z01 = _tc2(sp.reshape(NC, ROWS, 128), y, dinv,
               W1, b1.reshape(1, 16), W2.T)

    op0, op1 = _pair_pass(ei, z01.reshape(NC * NP), zeros)
    out0, out1 = _tc3(op0.reshape(NC, ROWS, 128), op1.reshape(NC, ROWS, 128),
                      z01, dinv, b2.reshape(1, 2))

    return jnp.stack([out0.reshape(NP)[:N], out1.reshape(NP)[:N]], axis=1)
